# Initial kernel scaffold; baseline (speedup 1.0000x reference)
import functools

import jax
import jax.numpy as jnp
from jax import lax
from jax.experimental import pallas as pl
from jax.experimental.pallas import tpu as pltpu
from jax.experimental.pallas import tpu_sc as plsc

N = 10000
E = 320000
G = 256
NEG = 0.01

KB = 80
ER = E // KB
NSC = 2
NT = 16
NPT = N // NT
RPT1 = ER // NSC // NT
RPT23 = ER // NT

RB = 2000
NB = N // RB


def _leaky(v):
    return jnp.where(v > 0, v, NEG * v)


def _dot_t(a, w):
    return lax.dot_general(a, w, (((1,), (1,)), ((), ())),
                           preferred_element_type=jnp.float32)


def _sc_mesh():
    return plsc.VectorSubcoreMesh(core_axis_name="c", subcore_axis_name="s")


def _sc_agg1(x, src2, dst2, zf, zd, ones):
    @functools.partial(
        pl.kernel,
        out_type=(jax.ShapeDtypeStruct((NSC, N, 128), jnp.float32),
                  jax.ShapeDtypeStruct((NSC, N, 16), jnp.float32)),
        mesh=_sc_mesh(),
        scratch_types=[
            pltpu.VMEM_SHARED((N, 128), jnp.float32),
            pltpu.VMEM_SHARED((N, 16), jnp.float32),
            pltpu.VMEM((RPT1, KB), jnp.int32),
            pltpu.VMEM((RPT1, KB), jnp.int32),
            pltpu.VMEM((KB, 128), jnp.float32),
            pltpu.VMEM((KB, 16), jnp.float32),
            pltpu.SemaphoreType.DMA,
        ],
    )
    def k(x_h, src_h, dst_h, zf_h, zd_h, on_h, sum_o, deg_o,
          acc, dacc, src_v, dst_v, rows_v, ones_v, sem):
        c = lax.axis_index("c")
        s = lax.axis_index("s")
        r0 = s * NPT
        pltpu.sync_copy(zf_h, acc.at[pl.ds(r0, NPT), :])
        pltpu.sync_copy(zd_h, dacc.at[pl.ds(r0, NPT), :])
        pltpu.sync_copy(on_h, ones_v)
        row0 = c * (ER // NSC) + s * RPT1
        pltpu.sync_copy(src_h.at[pl.ds(row0, RPT1), :], src_v)
        pltpu.sync_copy(dst_h.at[pl.ds(row0, RPT1), :], dst_v)
        plsc.subcore_barrier()

        def step(j, carry):
            pltpu.async_copy(x_h.at[src_v.at[j]], rows_v, sem).wait()
            pltpu.sync_copy(rows_v, acc.at[dst_v.at[j]], add=True)
            pltpu.sync_copy(ones_v, dacc.at[dst_v.at[j]], add=True)
            return carry

        lax.fori_loop(0, RPT1, step, 0)
        plsc.subcore_barrier()

        @pl.when(c == 0)
        def _():
            pltpu.sync_copy(acc.at[pl.ds(r0, NPT), :],
                            sum_o.at[0, pl.ds(r0, NPT), :])
            pltpu.sync_copy(dacc.at[pl.ds(r0, NPT), :],
                            deg_o.at[0, pl.ds(r0, NPT), :])

        @pl.when(c == 1)
        def _():
            pltpu.sync_copy(acc.at[pl.ds(r0, NPT), :],
                            sum_o.at[1, pl.ds(r0, NPT), :])
            pltpu.sync_copy(dacc.at[pl.ds(r0, NPT), :],
                            deg_o.at[1, pl.ds(r0, NPT), :])

    return k(x, src2, dst2, zf, zd, ones)


def _sc_agg23(h_parts, src2, dst2, zf):
    @functools.partial(
        pl.kernel,
        out_type=jax.ShapeDtypeStruct((NSC, N, 128), jnp.float32),
        mesh=_sc_mesh(),
        scratch_types=[
            pltpu.VMEM_SHARED((N, 128), jnp.float32),
            pltpu.VMEM((RPT23, KB), jnp.int32),
            pltpu.VMEM((RPT23, KB), jnp.int32),
            pltpu.VMEM((KB, 128), jnp.float32),
            pltpu.SemaphoreType.DMA,
        ],
    )
    def k(hp_h, src_h, dst_h, zf_h, sum_o, acc, src_v, dst_v, rows_v, sem):
        c = lax.axis_index("c")
        s = lax.axis_index("s")
        r0 = s * NPT
        pltpu.sync_copy(zf_h, acc.at[pl.ds(r0, NPT), :])
        row0 = s * RPT23
        pltpu.sync_copy(src_h.at[pl.ds(row0, RPT23), :], src_v)
        pltpu.sync_copy(dst_h.at[pl.ds(row0, RPT23), :], dst_v)
        plsc.subcore_barrier()

        def step(j, carry):
            @pl.when(c == 0)
            def _():
                pltpu.async_copy(hp_h.at[0].at[src_v.at[j]], rows_v, sem).wait()

            @pl.when(c == 1)
            def _():
                pltpu.async_copy(hp_h.at[1].at[src_v.at[j]], rows_v, sem).wait()

            pltpu.sync_copy(rows_v, acc.at[dst_v.at[j]], add=True)
            return carry

        lax.fori_loop(0, RPT23, step, 0)
        plsc.subcore_barrier()

        @pl.when(c == 0)
        def _():
            pltpu.sync_copy(acc.at[pl.ds(r0, NPT), :],
                            sum_o.at[0, pl.ds(r0, NPT), :])

        @pl.when(c == 1)
        def _():
            pltpu.sync_copy(acc.at[pl.ds(r0, NPT), :],
                            sum_o.at[1, pl.ds(r0, NPT), :])

    return k(h_parts, src2, dst2, zf)


def _inv_deg(dp):
    d = dp[0, :, 0:1] + dp[1, :, 0:1]
    return 1.0 / jnp.maximum(d, 1.0)


def _tc_layer1(sum_parts, deg_parts, x, wl, bl, wr):
    def body(sp, dp, xr, wl_r, bl_r, wr_r, out):
        mean = (sp[0] + sp[1]) * _inv_deg(dp)
        h = _dot_t(mean, wl_r[...]) + _dot_t(xr[...], wr_r[...]) + bl_r[...]
        h = _leaky(h)
        out[0] = h[:, :128]
        out[1] = h[:, 128:]

    return pl.pallas_call(
        body,
        grid=(NB,),
        in_specs=[
            pl.BlockSpec((NSC, RB, 128), lambda i: (0, i, 0)),
            pl.BlockSpec((NSC, RB, 16), lambda i: (0, i, 0)),
            pl.BlockSpec((RB, 128), lambda i: (i, 0)),
            pl.BlockSpec((256, 128), lambda i: (0, 0)),
            pl.BlockSpec((1, 256), lambda i: (0, 0)),
            pl.BlockSpec((256, 128), lambda i: (0, 0)),
        ],
        out_specs=pl.BlockSpec((NSC, RB, 128), lambda i: (0, i, 0)),
        out_shape=jax.ShapeDtypeStruct((NSC, N, 128), jnp.float32),
    )(sum_parts, deg_parts, x, wl, bl, wr)


def _tc_layer2(sum_parts, deg_parts, h_parts, wl, bl, wr):
    def body(sp, dp, hp, wl_r, bl_r, wr_r, out):
        mean = jnp.concatenate([sp[0], sp[1]], axis=1) * _inv_deg(dp)
        selfv = jnp.concatenate([hp[0], hp[1]], axis=1)
        h = _dot_t(mean, wl_r[...]) + _dot_t(selfv, wr_r[...]) + bl_r[...]
        h = _leaky(h)
        out[0] = h[:, :128]
        out[1] = h[:, 128:]

    return pl.pallas_call(
        body,
        grid=(NB,),
        in_specs=[
            pl.BlockSpec((NSC, RB, 128), lambda i: (0, i, 0)),
            pl.BlockSpec((NSC, RB, 16), lambda i: (0, i, 0)),
            pl.BlockSpec((NSC, RB, 128), lambda i: (0, i, 0)),
            pl.BlockSpec((256, 256), lambda i: (0, 0)),
            pl.BlockSpec((1, 256), lambda i: (0, 0)),
            pl.BlockSpec((256, 256), lambda i: (0, 0)),
        ],
        out_specs=pl.BlockSpec((NSC, RB, 128), lambda i: (0, i, 0)),
        out_shape=jax.ShapeDtypeStruct((NSC, N, 128), jnp.float32),
    )(sum_parts, deg_parts, h_parts, wl, bl, wr)


def _tc_layer3_pool_mlp(sum_parts, deg_parts, h_parts, wl, bl, wr, batch,
                        wg, bg, wf1, bf1, wf2, bf2):
    def body(sp, dp, hp, wl_r, bl_r, wr_r, batch_s, wg_r, bg_r,
             wf1_r, bf1_r, wf2_r, bf2_r, out, g_acc, h3_v):
        i = pl.program_id(0)

        @pl.when(i == 0)
        def _():
            g_acc[...] = jnp.full((G, 512), -jnp.inf, jnp.float32)

        mean = jnp.concatenate([sp[0], sp[1]], axis=1) * _inv_deg(dp)
        selfv = jnp.concatenate([hp[0], hp[1]], axis=1)
        h = _dot_t(mean, wl_r[...]) + _dot_t(selfv, wr_r[...]) + bl_r[...]
        h3_v[...] = _leaky(h)

        def row_step(r, carry):
            b_i = batch_s[r]
            row = h3_v[pl.ds(r, 1), :]
            cur = g_acc[pl.ds(b_i, 1), :]
            g_acc[pl.ds(b_i, 1), :] = jnp.maximum(cur, row)
            return carry

        lax.fori_loop(0, RB, row_step, 0)

        @pl.when(i == NB - 1)
        def _():
            g = g_acc[...]
            t = _dot_t(g, wg_r[...]) + bg_r[...]
            t = _leaky(_dot_t(t, wf1_r[...]) + bf1_r[...])
            out[...] = _dot_t(t, wf2_r[...]) + bf2_r[...]

    return pl.pallas_call(
        body,
        grid=(NB,),
        in_specs=[
            pl.BlockSpec((NSC, RB, 128), lambda i: (0, i, 0)),
            pl.BlockSpec((NSC, RB, 16), lambda i: (0, i, 0)),
            pl.BlockSpec((NSC, RB, 128), lambda i: (0, i, 0)),
            pl.BlockSpec((512, 256), lambda i: (0, 0)),
            pl.BlockSpec((1, 512), lambda i: (0, 0)),
            pl.BlockSpec((512, 256), lambda i: (0, 0)),
            pl.BlockSpec((RB,), lambda i: (i,), memory_space=pltpu.SMEM),
            pl.BlockSpec((128, 512), lambda i: (0, 0)),
            pl.BlockSpec((1, 128), lambda i: (0, 0)),
            pl.BlockSpec((128, 128), lambda i: (0, 0)),
            pl.BlockSpec((1, 128), lambda i: (0, 0)),
            pl.BlockSpec((2, 128), lambda i: (0, 0)),
            pl.BlockSpec((1, 2), lambda i: (0, 0)),
        ],
        out_specs=pl.BlockSpec((G, 2), lambda i: (0, 0)),
        out_shape=jax.ShapeDtypeStruct((G, 2), jnp.float32),
        scratch_shapes=[
            pltpu.VMEM((G, 512), jnp.float32),
            pltpu.VMEM((RB, 512), jnp.float32),
        ],
    )(sum_parts, deg_parts, h_parts, wl, bl, wr, batch,
      wg, bg, wf1, bf1, wf2, bf2)


def kernel(x, edge_index, batch, W_l1, b_l1, W_r1, W_l2, b_l2, W_r2,
           W_l3, b_l3, W_r3, W_g1, b_g1, W_f1, b_f1, W_f2, b_f2):
    src2 = edge_index[0].reshape(ER, KB)
    dst2 = edge_index[1].reshape(ER, KB)
    zf = jnp.zeros((NPT, 128), jnp.float32)
    zd = jnp.zeros((NPT, 16), jnp.float32)
    ones = jnp.ones((KB, 16), jnp.float32)

    sum1, deg = _sc_agg1(x, src2, dst2, zf, zd, ones)
    h1 = _tc_layer1(sum1, deg, x, W_l1, b_l1.reshape(1, -1), W_r1)
    sum2 = _sc_agg23(h1, src2, dst2, zf)
    h2 = _tc_layer2(sum2, deg, h1, W_l2, b_l2.reshape(1, -1), W_r2)
    sum3 = _sc_agg23(h2, src2, dst2, zf)
    out = _tc_layer3_pool_mlp(
        sum3, deg, h2, W_l3, b_l3.reshape(1, -1), W_r3, batch,
        W_g1, b_g1.reshape(1, -1), W_f1, b_f1.reshape(1, -1),
        W_f2, b_f2.reshape(1, -1))
    return out


# speedup vs baseline: 3.9973x; 3.9973x over previous
import functools

import jax
import jax.numpy as jnp
from jax import lax
from jax.experimental import pallas as pl
from jax.experimental.pallas import tpu as pltpu
from jax.experimental.pallas import tpu_sc as plsc

N = 10000
E = 320000
G = 256
NEG = 0.01

KB = 80
ER = E // KB
NSC = 2
NT = 16
NPAD = 10240
NPT = NPAD // NT
RPT1 = ER // NSC // NT
DCH = 25
DCB = RPT1 // DCH
RPT23 = ER // NT
ACH = 25
ACB = RPT23 // ACH

RB = 2000
NB = N // RB


def _leaky(v):
    return jnp.where(v > 0, v, NEG * v)


def _dot_t(a, w):
    return lax.dot_general(a, w, (((1,), (1,)), ((), ())),
                           preferred_element_type=jnp.float32)


def _sc_mesh():
    return plsc.VectorSubcoreMesh(core_axis_name="c", subcore_axis_name="s")


def _sc_deg(dst1, zd, ones):
    @functools.partial(
        pl.kernel,
        out_type=jax.ShapeDtypeStruct((NSC, NPAD, 128), jnp.float32),
        mesh=_sc_mesh(),
        scratch_types=[
            pltpu.VMEM_SHARED((NPAD, 128), jnp.float32),
            pltpu.VMEM((DCB, KB), jnp.int32),
            pltpu.VMEM((KB, 128), jnp.float32),
        ],
    )
    def k(dst_h, zd_h, on_h, deg_o, dacc, dst_v, ones_v):
        c = lax.axis_index("c")
        s = lax.axis_index("s")
        r0 = s * NPT
        pltpu.sync_copy(zd_h, dacc.at[pl.ds(r0, NPT), :])
        pltpu.sync_copy(on_h, ones_v)
        tid = c * NT + s
        plsc.subcore_barrier()

        def chunk(ch, carry):
            pltpu.sync_copy(dst_h.at[tid, ch], dst_v)

            def step(j, carry2):
                pltpu.sync_copy(ones_v, dacc.at[dst_v.at[j]], add=True)
                return carry2

            return lax.fori_loop(0, DCB, step, carry)

        lax.fori_loop(0, DCH, chunk, 0)
        plsc.subcore_barrier()

        @pl.when(c == 0)
        def _():
            pltpu.sync_copy(dacc.at[pl.ds(r0, NPT), :],
                            deg_o.at[0, pl.ds(r0, NPT), :])

        @pl.when(c == 1)
        def _():
            pltpu.sync_copy(dacc.at[pl.ds(r0, NPT), :],
                            deg_o.at[1, pl.ds(r0, NPT), :])

    return k(dst1, zd, ones)


def _sc_agg23(h_parts, src2, dst2, zf):
    @functools.partial(
        pl.kernel,
        out_type=jax.ShapeDtypeStruct((NSC, NPAD, 128), jnp.float32),
        mesh=_sc_mesh(),
        scratch_types=[
            pltpu.VMEM_SHARED((NPAD, 128), jnp.float32),
            pltpu.VMEM((ACB, KB), jnp.int32),
            pltpu.VMEM((ACB, KB), jnp.int32),
            pltpu.VMEM((KB, 128), jnp.float32),
            pltpu.SemaphoreType.DMA,
        ],
    )
    def k(hp_h, src_h, dst_h, zf_h, sum_o, acc, src_v, dst_v, rows_v, sem):
        c = lax.axis_index("c")
        s = lax.axis_index("s")
        r0 = s * NPT
        pltpu.sync_copy(zf_h, acc.at[pl.ds(r0, NPT), :])
        plsc.subcore_barrier()

        def chunk(ch, carry):
            pltpu.sync_copy(src_h.at[s, ch], src_v)
            pltpu.sync_copy(dst_h.at[s, ch], dst_v)

            def step(j, carry2):
                @pl.when(c == 0)
                def _():
                    pltpu.async_copy(hp_h.at[0].at[src_v.at[j]], rows_v,
                                     sem).wait()

                @pl.when(c == 1)
                def _():
                    pltpu.async_copy(hp_h.at[1].at[src_v.at[j]], rows_v,
                                     sem).wait()

                pltpu.sync_copy(rows_v, acc.at[dst_v.at[j]], add=True)
                return carry2

            return lax.fori_loop(0, ACB, step, carry)

        lax.fori_loop(0, ACH, chunk, 0)
        plsc.subcore_barrier()

        @pl.when(c == 0)
        def _():
            pltpu.sync_copy(acc.at[pl.ds(r0, NPT), :],
                            sum_o.at[0, pl.ds(r0, NPT), :])

        @pl.when(c == 1)
        def _():
            pltpu.sync_copy(acc.at[pl.ds(r0, NPT), :],
                            sum_o.at[1, pl.ds(r0, NPT), :])

    return k(h_parts, src2, dst2, zf)


def _inv_deg(dp):
    d = dp[0, :, 0:1] + dp[1, :, 0:1]
    return 1.0 / jnp.maximum(d, 1.0)


def _tc_layer1(sum_parts, deg_parts, x, wl, bl, wr):
    def body(sp, dp, xr, wl_r, bl_r, wr_r, out):
        mean = sp[0] * _inv_deg(dp)
        h = _dot_t(mean, wl_r[...]) + _dot_t(xr[...], wr_r[...]) + bl_r[...]
        h = _leaky(h)
        out[0] = h[:, :128]
        out[1] = h[:, 128:]

    return pl.pallas_call(
        body,
        grid=(NB,),
        in_specs=[
            pl.BlockSpec((NSC, RB, 128), lambda i: (0, i, 0)),
            pl.BlockSpec((NSC, RB, 128), lambda i: (0, i, 0)),
            pl.BlockSpec((RB, 128), lambda i: (i, 0)),
            pl.BlockSpec((256, 128), lambda i: (0, 0)),
            pl.BlockSpec((1, 256), lambda i: (0, 0)),
            pl.BlockSpec((256, 128), lambda i: (0, 0)),
        ],
        out_specs=pl.BlockSpec((NSC, RB, 128), lambda i: (0, i, 0)),
        out_shape=jax.ShapeDtypeStruct((NSC, N, 128), jnp.float32),
    )(sum_parts, deg_parts, x, wl, bl, wr)


def _tc_layer2(sum_parts, deg_parts, h_parts, wl, bl, wr):
    def body(sp, dp, hp, wl_r, bl_r, wr_r, out):
        mean = jnp.concatenate([sp[0], sp[1]], axis=1) * _inv_deg(dp)
        selfv = jnp.concatenate([hp[0], hp[1]], axis=1)
        h = _dot_t(mean, wl_r[...]) + _dot_t(selfv, wr_r[...]) + bl_r[...]
        h = _leaky(h)
        out[0] = h[:, :128]
        out[1] = h[:, 128:]

    return pl.pallas_call(
        body,
        grid=(NB,),
        in_specs=[
            pl.BlockSpec((NSC, RB, 128), lambda i: (0, i, 0)),
            pl.BlockSpec((NSC, RB, 128), lambda i: (0, i, 0)),
            pl.BlockSpec((NSC, RB, 128), lambda i: (0, i, 0)),
            pl.BlockSpec((256, 256), lambda i: (0, 0)),
            pl.BlockSpec((1, 256), lambda i: (0, 0)),
            pl.BlockSpec((256, 256), lambda i: (0, 0)),
        ],
        out_specs=pl.BlockSpec((NSC, RB, 128), lambda i: (0, i, 0)),
        out_shape=jax.ShapeDtypeStruct((NSC, N, 128), jnp.float32),
    )(sum_parts, deg_parts, h_parts, wl, bl, wr)


def _tc_layer3_pool_mlp(sum_parts, deg_parts, h_parts, wl, bl, wr, batch,
                        wg, bg, wf1, bf1, wf2, bf2):
    def body(sp, dp, hp, wl_r, bl_r, wr_r, batch_s, wg_r, bg_r,
             wf1_r, bf1_r, wf2_r, bf2_r, out, g_acc, h3_v):
        i = pl.program_id(0)

        @pl.when(i == 0)
        def _():
            g_acc[...] = jnp.full((G, 512), -jnp.inf, jnp.float32)

        mean = jnp.concatenate([sp[0], sp[1]], axis=1) * _inv_deg(dp)
        selfv = jnp.concatenate([hp[0], hp[1]], axis=1)
        h = _dot_t(mean, wl_r[...]) + _dot_t(selfv, wr_r[...]) + bl_r[...]
        h3_v[...] = _leaky(h)

        def row_step(r, carry):
            b_i = batch_s[i * RB + r]
            row = h3_v[pl.ds(r, 1), :]
            cur = g_acc[pl.ds(b_i, 1), :]
            g_acc[pl.ds(b_i, 1), :] = jnp.maximum(cur, row)
            return carry

        lax.fori_loop(0, RB, row_step, 0)

        @pl.when(i == NB - 1)
        def _():
            g = g_acc[...]
            t = _dot_t(g, wg_r[...]) + bg_r[...]
            t = _leaky(_dot_t(t, wf1_r[...]) + bf1_r[...])
            out[...] = _dot_t(t, wf2_r[...]) + bf2_r[...]

    return pl.pallas_call(
        body,
        grid=(NB,),
        in_specs=[
            pl.BlockSpec((NSC, RB, 128), lambda i: (0, i, 0)),
            pl.BlockSpec((NSC, RB, 128), lambda i: (0, i, 0)),
            pl.BlockSpec((NSC, RB, 128), lambda i: (0, i, 0)),
            pl.BlockSpec((512, 256), lambda i: (0, 0)),
            pl.BlockSpec((1, 512), lambda i: (0, 0)),
            pl.BlockSpec((512, 256), lambda i: (0, 0)),
            pl.BlockSpec((N,), lambda i: (0,), memory_space=pltpu.SMEM),
            pl.BlockSpec((128, 512), lambda i: (0, 0)),
            pl.BlockSpec((1, 128), lambda i: (0, 0)),
            pl.BlockSpec((128, 128), lambda i: (0, 0)),
            pl.BlockSpec((1, 128), lambda i: (0, 0)),
            pl.BlockSpec((2, 128), lambda i: (0, 0)),
            pl.BlockSpec((1, 2), lambda i: (0, 0)),
        ],
        out_specs=pl.BlockSpec((G, 2), lambda i: (0, 0)),
        out_shape=jax.ShapeDtypeStruct((G, 2), jnp.float32),
        scratch_shapes=[
            pltpu.VMEM((G, 512), jnp.float32),
            pltpu.VMEM((RB, 512), jnp.float32),
        ],
    )(sum_parts, deg_parts, h_parts, wl, bl, wr, batch,
      wg, bg, wf1, bf1, wf2, bf2)


def kernel(x, edge_index, batch, W_l1, b_l1, W_r1, W_l2, b_l2, W_r2,
           W_l3, b_l3, W_r3, W_g1, b_g1, W_f1, b_f1, W_f2, b_f2):
    dst1 = edge_index[1].reshape(NSC * NT, DCH, DCB, KB)
    srcf = edge_index[0].reshape(NT, ACH, ACB, KB)
    dstf = edge_index[1].reshape(NT, ACH, ACB, KB)
    zf = jnp.zeros((NPT, 128), jnp.float32)
    ones = jnp.ones((KB, 128), jnp.float32)

    deg = _sc_deg(dst1, zf, ones)
    x2 = jnp.stack([x, x])
    sum1 = _sc_agg23(x2, srcf, dstf, zf)
    h1 = _tc_layer1(sum1, deg, x, W_l1, b_l1.reshape(1, -1), W_r1)
    sum2 = _sc_agg23(h1, srcf, dstf, zf)
    h2 = _tc_layer2(sum2, deg, h1, W_l2, b_l2.reshape(1, -1), W_r2)
    sum3 = _sc_agg23(h2, srcf, dstf, zf)
    out = _tc_layer3_pool_mlp(
        sum3, deg, h2, W_l3, b_l3.reshape(1, -1), W_r3, batch,
        W_g1, b_g1.reshape(1, -1), W_f1, b_f1.reshape(1, -1),
        W_f2, b_f2.reshape(1, -1))
    return out


# speedup vs baseline: 5.6828x; 1.4216x over previous
import functools

import jax
import jax.numpy as jnp
from jax import lax
from jax.experimental import pallas as pl
from jax.experimental.pallas import tpu as pltpu
from jax.experimental.pallas import tpu_sc as plsc

N = 10000
E = 320000
G = 256
NEG = 0.01

KB = 80
ER = E // KB
NSC = 2
NT = 16
NPAD = 10240
NPT = NPAD // NT
RPT1 = ER // NSC // NT
DCH = 25
DCB = RPT1 // DCH
RPT23 = ER // NT
ACH = 25
ACB = RPT23 // ACH

RB = 2000
NB = N // RB


def _leaky(v):
    return jnp.where(v > 0, v, NEG * v)


def _dot_t(a, w):
    return lax.dot_general(a, w, (((1,), (1,)), ((), ())),
                           preferred_element_type=jnp.float32)


def _sc_mesh():
    return plsc.VectorSubcoreMesh(core_axis_name="c", subcore_axis_name="s")


def _sc_deg(dst1, zd, ones):
    @functools.partial(
        pl.kernel,
        out_type=jax.ShapeDtypeStruct((NSC, NPAD, 128), jnp.float32),
        mesh=_sc_mesh(),
        scratch_types=[
            pltpu.VMEM_SHARED((NPAD, 128), jnp.float32),
            pltpu.VMEM((DCB, KB), jnp.int32),
            pltpu.VMEM((KB, 128), jnp.float32),
        ],
    )
    def k(dst_h, zd_h, on_h, deg_o, dacc, dst_v, ones_v):
        c = lax.axis_index("c")
        s = lax.axis_index("s")
        r0 = s * NPT
        pltpu.sync_copy(zd_h, dacc.at[pl.ds(r0, NPT), :])
        pltpu.sync_copy(on_h, ones_v)
        tid = c * NT + s
        plsc.subcore_barrier()

        def chunk(ch, carry):
            pltpu.sync_copy(dst_h.at[tid, ch], dst_v)

            def step(j, carry2):
                pltpu.sync_copy(ones_v, dacc.at[dst_v.at[j]], add=True)
                return carry2

            return lax.fori_loop(0, DCB, step, carry)

        lax.fori_loop(0, DCH, chunk, 0)
        plsc.subcore_barrier()

        @pl.when(c == 0)
        def _():
            pltpu.sync_copy(dacc.at[pl.ds(r0, NPT), :],
                            deg_o.at[0, pl.ds(r0, NPT), :])

        @pl.when(c == 1)
        def _():
            pltpu.sync_copy(dacc.at[pl.ds(r0, NPT), :],
                            deg_o.at[1, pl.ds(r0, NPT), :])

    return k(dst1, zd, ones)


def _sc_agg23(h_parts, src2, dst2, zf):
    @functools.partial(
        pl.kernel,
        out_type=jax.ShapeDtypeStruct((NSC, NPAD, 128), jnp.float32),
        mesh=_sc_mesh(),
        scratch_types=[
            pltpu.VMEM_SHARED((NPAD, 128), jnp.float32),
            pltpu.VMEM((ACB, KB), jnp.int32),
            pltpu.VMEM((ACB, KB), jnp.int32),
            pltpu.VMEM((KB, 128), jnp.float32),
            pltpu.VMEM((KB, 128), jnp.float32),
            pltpu.SemaphoreType.DMA,
            pltpu.SemaphoreType.DMA,
        ],
    )
    def k(hp_h, src_h, dst_h, zf_h, sum_o, acc, src_v, dst_v, rows0, rows1,
          sem0, sem1):
        c = lax.axis_index("c")
        s = lax.axis_index("s")
        r0 = s * NPT
        pltpu.sync_copy(zf_h, acc.at[pl.ds(r0, NPT), :])
        plsc.subcore_barrier()

        rows = (rows0, rows1)
        sems = (sem0, sem1)

        def issue(j, b):
            @pl.when(c == 0)
            def _():
                pltpu.async_copy(hp_h.at[0].at[src_v.at[j]], rows[b],
                                 sems[b])

            @pl.when(c == 1)
            def _():
                pltpu.async_copy(hp_h.at[1].at[src_v.at[j]], rows[b],
                                 sems[b])

        def drain_scatter(j, b):
            pltpu.make_async_copy(hp_h.at[0].at[src_v.at[j]], rows[b],
                                  sems[b]).wait()
            pltpu.sync_copy(rows[b], acc.at[dst_v.at[j]], add=True)

        def chunk(ch, carry):
            pltpu.sync_copy(src_h.at[s, ch], src_v)
            pltpu.sync_copy(dst_h.at[s, ch], dst_v)
            issue(0, 0)
            for j in range(1, ACB):
                issue(j, j % 2)
                drain_scatter(j - 1, (j - 1) % 2)
            drain_scatter(ACB - 1, (ACB - 1) % 2)
            return carry

        lax.fori_loop(0, ACH, chunk, 0)
        plsc.subcore_barrier()

        @pl.when(c == 0)
        def _():
            pltpu.sync_copy(acc.at[pl.ds(r0, NPT), :],
                            sum_o.at[0, pl.ds(r0, NPT), :])

        @pl.when(c == 1)
        def _():
            pltpu.sync_copy(acc.at[pl.ds(r0, NPT), :],
                            sum_o.at[1, pl.ds(r0, NPT), :])

    return k(h_parts, src2, dst2, zf)


def _inv_deg(dp):
    d = dp[0, :, 0:1] + dp[1, :, 0:1]
    return 1.0 / jnp.maximum(d, 1.0)


def _tc_layer1(sum_parts, deg_parts, x, wl, bl, wr):
    def body(sp, dp, xr, wl_r, bl_r, wr_r, out):
        mean = sp[0] * _inv_deg(dp)
        h = _dot_t(mean, wl_r[...]) + _dot_t(xr[...], wr_r[...]) + bl_r[...]
        h = _leaky(h)
        out[0] = h[:, :128]
        out[1] = h[:, 128:]

    return pl.pallas_call(
        body,
        grid=(NB,),
        in_specs=[
            pl.BlockSpec((NSC, RB, 128), lambda i: (0, i, 0)),
            pl.BlockSpec((NSC, RB, 128), lambda i: (0, i, 0)),
            pl.BlockSpec((RB, 128), lambda i: (i, 0)),
            pl.BlockSpec((256, 128), lambda i: (0, 0)),
            pl.BlockSpec((1, 256), lambda i: (0, 0)),
            pl.BlockSpec((256, 128), lambda i: (0, 0)),
        ],
        out_specs=pl.BlockSpec((NSC, RB, 128), lambda i: (0, i, 0)),
        out_shape=jax.ShapeDtypeStruct((NSC, N, 128), jnp.float32),
    )(sum_parts, deg_parts, x, wl, bl, wr)


def _tc_layer2(sum_parts, deg_parts, h_parts, wl, bl, wr):
    def body(sp, dp, hp, wl_r, bl_r, wr_r, out):
        mean = jnp.concatenate([sp[0], sp[1]], axis=1) * _inv_deg(dp)
        selfv = jnp.concatenate([hp[0], hp[1]], axis=1)
        h = _dot_t(mean, wl_r[...]) + _dot_t(selfv, wr_r[...]) + bl_r[...]
        h = _leaky(h)
        out[0] = h[:, :128]
        out[1] = h[:, 128:]

    return pl.pallas_call(
        body,
        grid=(NB,),
        in_specs=[
            pl.BlockSpec((NSC, RB, 128), lambda i: (0, i, 0)),
            pl.BlockSpec((NSC, RB, 128), lambda i: (0, i, 0)),
            pl.BlockSpec((NSC, RB, 128), lambda i: (0, i, 0)),
            pl.BlockSpec((256, 256), lambda i: (0, 0)),
            pl.BlockSpec((1, 256), lambda i: (0, 0)),
            pl.BlockSpec((256, 256), lambda i: (0, 0)),
        ],
        out_specs=pl.BlockSpec((NSC, RB, 128), lambda i: (0, i, 0)),
        out_shape=jax.ShapeDtypeStruct((NSC, N, 128), jnp.float32),
    )(sum_parts, deg_parts, h_parts, wl, bl, wr)


def _tc_layer3_pool_mlp(sum_parts, deg_parts, h_parts, wl, bl, wr, batch,
                        wg, bg, wf1, bf1, wf2, bf2):
    def body(sp, dp, hp, wl_r, bl_r, wr_r, batch_s, wg_r, bg_r,
             wf1_r, bf1_r, wf2_r, bf2_r, out, g_acc, h3_v):
        i = pl.program_id(0)

        @pl.when(i == 0)
        def _():
            g_acc[...] = jnp.full((G, 512), -jnp.inf, jnp.float32)

        mean = jnp.concatenate([sp[0], sp[1]], axis=1) * _inv_deg(dp)
        selfv = jnp.concatenate([hp[0], hp[1]], axis=1)
        h = _dot_t(mean, wl_r[...]) + _dot_t(selfv, wr_r[...]) + bl_r[...]
        h3_v[...] = _leaky(h)

        def row_step(r, carry):
            b_i = batch_s[i * RB + r]
            row = h3_v[pl.ds(r, 1), :]
            cur = g_acc[pl.ds(b_i, 1), :]
            g_acc[pl.ds(b_i, 1), :] = jnp.maximum(cur, row)
            return carry

        lax.fori_loop(0, RB, row_step, 0)

        @pl.when(i == NB - 1)
        def _():
            g = g_acc[...]
            t = _dot_t(g, wg_r[...]) + bg_r[...]
            t = _leaky(_dot_t(t, wf1_r[...]) + bf1_r[...])
            out[...] = _dot_t(t, wf2_r[...]) + bf2_r[...]

    return pl.pallas_call(
        body,
        grid=(NB,),
        in_specs=[
            pl.BlockSpec((NSC, RB, 128), lambda i: (0, i, 0)),
            pl.BlockSpec((NSC, RB, 128), lambda i: (0, i, 0)),
            pl.BlockSpec((NSC, RB, 128), lambda i: (0, i, 0)),
            pl.BlockSpec((512, 256), lambda i: (0, 0)),
            pl.BlockSpec((1, 512), lambda i: (0, 0)),
            pl.BlockSpec((512, 256), lambda i: (0, 0)),
            pl.BlockSpec((N,), lambda i: (0,), memory_space=pltpu.SMEM),
            pl.BlockSpec((128, 512), lambda i: (0, 0)),
            pl.BlockSpec((1, 128), lambda i: (0, 0)),
            pl.BlockSpec((128, 128), lambda i: (0, 0)),
            pl.BlockSpec((1, 128), lambda i: (0, 0)),
            pl.BlockSpec((2, 128), lambda i: (0, 0)),
            pl.BlockSpec((1, 2), lambda i: (0, 0)),
        ],
        out_specs=pl.BlockSpec((G, 2), lambda i: (0, 0)),
        out_shape=jax.ShapeDtypeStruct((G, 2), jnp.float32),
        scratch_shapes=[
            pltpu.VMEM((G, 512), jnp.float32),
            pltpu.VMEM((RB, 512), jnp.float32),
        ],
    )(sum_parts, deg_parts, h_parts, wl, bl, wr, batch,
      wg, bg, wf1, bf1, wf2, bf2)


def kernel(x, edge_index, batch, W_l1, b_l1, W_r1, W_l2, b_l2, W_r2,
           W_l3, b_l3, W_r3, W_g1, b_g1, W_f1, b_f1, W_f2, b_f2):
    dst1 = edge_index[1].reshape(NSC * NT, DCH, DCB, KB)
    srcf = edge_index[0].reshape(NT, ACH, ACB, KB)
    dstf = edge_index[1].reshape(NT, ACH, ACB, KB)
    zf = jnp.zeros((NPT, 128), jnp.float32)
    ones = jnp.ones((KB, 128), jnp.float32)

    deg = _sc_deg(dst1, zf, ones)
    x2 = jnp.stack([x, x])
    sum1 = _sc_agg23(x2, srcf, dstf, zf)
    h1 = _tc_layer1(sum1, deg, x, W_l1, b_l1.reshape(1, -1), W_r1)
    sum2 = _sc_agg23(h1, srcf, dstf, zf)
    h2 = _tc_layer2(sum2, deg, h1, W_l2, b_l2.reshape(1, -1), W_r2)
    sum3 = _sc_agg23(h2, srcf, dstf, zf)
    out = _tc_layer3_pool_mlp(
        sum3, deg, h2, W_l3, b_l3.reshape(1, -1), W_r3, batch,
        W_g1, b_g1.reshape(1, -1), W_f1, b_f1.reshape(1, -1),
        W_f2, b_f2.reshape(1, -1))
    return out


# speedup vs baseline: 6.1342x; 1.0794x over previous
import functools

import jax
import jax.numpy as jnp
from jax import lax
from jax.experimental import pallas as pl
from jax.experimental.pallas import tpu as pltpu
from jax.experimental.pallas import tpu_sc as plsc

N = 10000
E = 320000
G = 256
NEG = 0.01

KB = 80
ER = E // KB
NSC = 2
NT = 16
NPAD = 10240
NPT = NPAD // NT
RPT1 = ER // NSC // NT
DCH = 25
DCB = RPT1 // DCH
RPT23 = ER // NT
ACH = 25
ACB = RPT23 // ACH

RB = 2000
NB = N // RB


def _leaky(v):
    return jnp.where(v > 0, v, NEG * v)


def _dot_t(a, w):
    return lax.dot_general(a, w, (((1,), (1,)), ((), ())),
                           preferred_element_type=jnp.float32)


def _sc_mesh():
    return plsc.VectorSubcoreMesh(core_axis_name="c", subcore_axis_name="s")


def _sc_deg(dst1, zd, ones):
    @functools.partial(
        pl.kernel,
        out_type=jax.ShapeDtypeStruct((NSC, NPAD, 128), jnp.float32),
        mesh=_sc_mesh(),
        scratch_types=[
            pltpu.VMEM_SHARED((NPAD, 128), jnp.float32),
            pltpu.VMEM((DCB, KB), jnp.int32),
            pltpu.VMEM((KB, 128), jnp.float32),
        ],
    )
    def k(dst_h, zd_h, on_h, deg_o, dacc, dst_v, ones_v):
        c = lax.axis_index("c")
        s = lax.axis_index("s")
        r0 = s * NPT
        pltpu.sync_copy(zd_h, dacc.at[pl.ds(r0, NPT), :])
        pltpu.sync_copy(on_h, ones_v)
        tid = c * NT + s
        plsc.subcore_barrier()

        def chunk(ch, carry):
            pltpu.sync_copy(dst_h.at[tid, ch], dst_v)

            def step(j, carry2):
                pltpu.sync_copy(ones_v, dacc.at[dst_v.at[j]], add=True)
                return carry2

            return lax.fori_loop(0, DCB, step, carry)

        lax.fori_loop(0, DCH, chunk, 0)
        plsc.subcore_barrier()

        @pl.when(c == 0)
        def _():
            pltpu.sync_copy(dacc.at[pl.ds(r0, NPT), :],
                            deg_o.at[0, pl.ds(r0, NPT), :])

        @pl.when(c == 1)
        def _():
            pltpu.sync_copy(dacc.at[pl.ds(r0, NPT), :],
                            deg_o.at[1, pl.ds(r0, NPT), :])

    return k(dst1, zd, ones)


def _sc_agg23(h_parts, src2, dst2, zf):
    @functools.partial(
        pl.kernel,
        out_type=jax.ShapeDtypeStruct((NSC, NPAD, 128), jnp.float32),
        mesh=_sc_mesh(),
        scratch_types=[
            pltpu.VMEM_SHARED((NPAD, 128), jnp.float32),
            pltpu.VMEM((ACB, KB), jnp.int32),
            pltpu.VMEM((ACB, KB), jnp.int32),
            pltpu.VMEM((KB, 128), jnp.float32),
            pltpu.VMEM((KB, 128), jnp.float32),
            pltpu.VMEM((KB, 128), jnp.float32),
            pltpu.SemaphoreType.DMA,
            pltpu.SemaphoreType.DMA,
            pltpu.SemaphoreType.DMA,
        ],
    )
    def k(hp_h, src_h, dst_h, zf_h, sum_o, acc, src_v, dst_v, rows0, rows1,
          rows2, sem0, sem1, sem2):
        c = lax.axis_index("c")
        s = lax.axis_index("s")
        r0 = s * NPT
        pltpu.sync_copy(zf_h, acc.at[pl.ds(r0, NPT), :])
        plsc.subcore_barrier()

        rows = (rows0, rows1, rows2)
        sems = (sem0, sem1, sem2)
        nbuf = len(rows)
        depth = nbuf - 1

        def issue(j, b):
            @pl.when(c == 0)
            def _():
                pltpu.async_copy(hp_h.at[0].at[src_v.at[j]], rows[b],
                                 sems[b])

            @pl.when(c == 1)
            def _():
                pltpu.async_copy(hp_h.at[1].at[src_v.at[j]], rows[b],
                                 sems[b])

        def drain_scatter(j, b):
            pltpu.make_async_copy(hp_h.at[0].at[src_v.at[j]], rows[b],
                                  sems[b]).wait()
            pltpu.sync_copy(rows[b], acc.at[dst_v.at[j]], add=True)

        def chunk(ch, carry):
            pltpu.sync_copy(src_h.at[s, ch], src_v)
            pltpu.sync_copy(dst_h.at[s, ch], dst_v)
            for j in range(depth):
                issue(j, j % nbuf)
            for j in range(ACB):
                if j + depth < ACB:
                    issue(j + depth, (j + depth) % nbuf)
                drain_scatter(j, j % nbuf)
            return carry

        lax.fori_loop(0, ACH, chunk, 0)
        plsc.subcore_barrier()

        @pl.when(c == 0)
        def _():
            pltpu.sync_copy(acc.at[pl.ds(r0, NPT), :],
                            sum_o.at[0, pl.ds(r0, NPT), :])

        @pl.when(c == 1)
        def _():
            pltpu.sync_copy(acc.at[pl.ds(r0, NPT), :],
                            sum_o.at[1, pl.ds(r0, NPT), :])

    return k(h_parts, src2, dst2, zf)


def _inv_deg(dp):
    d = dp[0, :, 0:1] + dp[1, :, 0:1]
    return 1.0 / jnp.maximum(d, 1.0)


def _tc_layer1(sum_parts, deg_parts, x, wl, bl, wr):
    def body(sp, dp, xr, wl_r, bl_r, wr_r, out):
        mean = sp[0] * _inv_deg(dp)
        h = _dot_t(mean, wl_r[...]) + _dot_t(xr[...], wr_r[...]) + bl_r[...]
        h = _leaky(h)
        out[0] = h[:, :128]
        out[1] = h[:, 128:]

    return pl.pallas_call(
        body,
        grid=(NB,),
        in_specs=[
            pl.BlockSpec((NSC, RB, 128), lambda i: (0, i, 0)),
            pl.BlockSpec((NSC, RB, 128), lambda i: (0, i, 0)),
            pl.BlockSpec((RB, 128), lambda i: (i, 0)),
            pl.BlockSpec((256, 128), lambda i: (0, 0)),
            pl.BlockSpec((1, 256), lambda i: (0, 0)),
            pl.BlockSpec((256, 128), lambda i: (0, 0)),
        ],
        out_specs=pl.BlockSpec((NSC, RB, 128), lambda i: (0, i, 0)),
        out_shape=jax.ShapeDtypeStruct((NSC, N, 128), jnp.float32),
    )(sum_parts, deg_parts, x, wl, bl, wr)


def _tc_layer2(sum_parts, deg_parts, h_parts, wl, bl, wr):
    def body(sp, dp, hp, wl_r, bl_r, wr_r, out):
        mean = jnp.concatenate([sp[0], sp[1]], axis=1) * _inv_deg(dp)
        selfv = jnp.concatenate([hp[0], hp[1]], axis=1)
        h = _dot_t(mean, wl_r[...]) + _dot_t(selfv, wr_r[...]) + bl_r[...]
        h = _leaky(h)
        out[0] = h[:, :128]
        out[1] = h[:, 128:]

    return pl.pallas_call(
        body,
        grid=(NB,),
        in_specs=[
            pl.BlockSpec((NSC, RB, 128), lambda i: (0, i, 0)),
            pl.BlockSpec((NSC, RB, 128), lambda i: (0, i, 0)),
            pl.BlockSpec((NSC, RB, 128), lambda i: (0, i, 0)),
            pl.BlockSpec((256, 256), lambda i: (0, 0)),
            pl.BlockSpec((1, 256), lambda i: (0, 0)),
            pl.BlockSpec((256, 256), lambda i: (0, 0)),
        ],
        out_specs=pl.BlockSpec((NSC, RB, 128), lambda i: (0, i, 0)),
        out_shape=jax.ShapeDtypeStruct((NSC, N, 128), jnp.float32),
    )(sum_parts, deg_parts, h_parts, wl, bl, wr)


def _tc_layer3_pool_mlp(sum_parts, deg_parts, h_parts, wl, bl, wr, batch,
                        wg, bg, wf1, bf1, wf2, bf2):
    def body(sp, dp, hp, wl_r, bl_r, wr_r, batch_s, wg_r, bg_r,
             wf1_r, bf1_r, wf2_r, bf2_r, out, g_acc, h3_v):
        i = pl.program_id(0)

        @pl.when(i == 0)
        def _():
            g_acc[...] = jnp.full((G, 512), -jnp.inf, jnp.float32)

        mean = jnp.concatenate([sp[0], sp[1]], axis=1) * _inv_deg(dp)
        selfv = jnp.concatenate([hp[0], hp[1]], axis=1)
        h = _dot_t(mean, wl_r[...]) + _dot_t(selfv, wr_r[...]) + bl_r[...]
        h3_v[...] = _leaky(h)

        def row_step(r, carry):
            b_i = batch_s[i * RB + r]
            row = h3_v[pl.ds(r, 1), :]
            cur = g_acc[pl.ds(b_i, 1), :]
            g_acc[pl.ds(b_i, 1), :] = jnp.maximum(cur, row)
            return carry

        lax.fori_loop(0, RB, row_step, 0)

        @pl.when(i == NB - 1)
        def _():
            g = g_acc[...]
            t = _dot_t(g, wg_r[...]) + bg_r[...]
            t = _leaky(_dot_t(t, wf1_r[...]) + bf1_r[...])
            out[...] = _dot_t(t, wf2_r[...]) + bf2_r[...]

    return pl.pallas_call(
        body,
        grid=(NB,),
        in_specs=[
            pl.BlockSpec((NSC, RB, 128), lambda i: (0, i, 0)),
            pl.BlockSpec((NSC, RB, 128), lambda i: (0, i, 0)),
            pl.BlockSpec((NSC, RB, 128), lambda i: (0, i, 0)),
            pl.BlockSpec((512, 256), lambda i: (0, 0)),
            pl.BlockSpec((1, 512), lambda i: (0, 0)),
            pl.BlockSpec((512, 256), lambda i: (0, 0)),
            pl.BlockSpec((N,), lambda i: (0,), memory_space=pltpu.SMEM),
            pl.BlockSpec((128, 512), lambda i: (0, 0)),
            pl.BlockSpec((1, 128), lambda i: (0, 0)),
            pl.BlockSpec((128, 128), lambda i: (0, 0)),
            pl.BlockSpec((1, 128), lambda i: (0, 0)),
            pl.BlockSpec((2, 128), lambda i: (0, 0)),
            pl.BlockSpec((1, 2), lambda i: (0, 0)),
        ],
        out_specs=pl.BlockSpec((G, 2), lambda i: (0, 0)),
        out_shape=jax.ShapeDtypeStruct((G, 2), jnp.float32),
        scratch_shapes=[
            pltpu.VMEM((G, 512), jnp.float32),
            pltpu.VMEM((RB, 512), jnp.float32),
        ],
    )(sum_parts, deg_parts, h_parts, wl, bl, wr, batch,
      wg, bg, wf1, bf1, wf2, bf2)


def kernel(x, edge_index, batch, W_l1, b_l1, W_r1, W_l2, b_l2, W_r2,
           W_l3, b_l3, W_r3, W_g1, b_g1, W_f1, b_f1, W_f2, b_f2):
    dst1 = edge_index[1].reshape(NSC * NT, DCH, DCB, KB)
    srcf = edge_index[0].reshape(NT, ACH, ACB, KB)
    dstf = edge_index[1].reshape(NT, ACH, ACB, KB)
    zf = jnp.zeros((NPT, 128), jnp.float32)
    ones = jnp.ones((KB, 128), jnp.float32)

    deg = _sc_deg(dst1, zf, ones)
    x2 = jnp.stack([x, x])
    sum1 = _sc_agg23(x2, srcf, dstf, zf)
    h1 = _tc_layer1(sum1, deg, x, W_l1, b_l1.reshape(1, -1), W_r1)
    sum2 = _sc_agg23(h1, srcf, dstf, zf)
    h2 = _tc_layer2(sum2, deg, h1, W_l2, b_l2.reshape(1, -1), W_r2)
    sum3 = _sc_agg23(h2, srcf, dstf, zf)
    out = _tc_layer3_pool_mlp(
        sum3, deg, h2, W_l3, b_l3.reshape(1, -1), W_r3, batch,
        W_g1, b_g1.reshape(1, -1), W_f1, b_f1.reshape(1, -1),
        W_f2, b_f2.reshape(1, -1))
    return out


# speedup vs baseline: 6.7010x; 1.0924x over previous
import functools

import jax
import jax.numpy as jnp
from jax import lax
from jax.experimental import pallas as pl
from jax.experimental.pallas import tpu as pltpu
from jax.experimental.pallas import tpu_sc as plsc

N = 10000
E = 320000
G = 256
NEG = 0.01

KB = 80
ER = E // KB
NSC = 2
NT = 16
NPAD = 10240
NPT = NPAD // NT
RPT1 = ER // NSC // NT
DCH = 25
DCB = RPT1 // DCH
RPT23 = ER // NT
ACH = 25
ACB = RPT23 // ACH

RB = 2000
NB = N // RB


def _leaky(v):
    return jnp.where(v > 0, v, NEG * v)


def _dot_t(a, w):
    return lax.dot_general(a, w, (((1,), (1,)), ((), ())),
                           preferred_element_type=jnp.float32)


def _sc_mesh():
    return plsc.VectorSubcoreMesh(core_axis_name="c", subcore_axis_name="s")


def _sc_deg(dst1, zd, ones):
    @functools.partial(
        pl.kernel,
        out_type=jax.ShapeDtypeStruct((NSC, NPAD, 128), jnp.float32),
        mesh=_sc_mesh(),
        scratch_types=[
            pltpu.VMEM_SHARED((NPAD, 128), jnp.float32),
            pltpu.VMEM((DCB, KB), jnp.int32),
            pltpu.VMEM((KB, 128), jnp.float32),
        ],
    )
    def k(dst_h, zd_h, on_h, deg_o, dacc, dst_v, ones_v):
        c = lax.axis_index("c")
        s = lax.axis_index("s")
        r0 = s * NPT
        pltpu.sync_copy(zd_h, dacc.at[pl.ds(r0, NPT), :])
        pltpu.sync_copy(on_h, ones_v)
        tid = c * NT + s
        plsc.subcore_barrier()

        def chunk(ch, carry):
            pltpu.sync_copy(dst_h.at[tid, ch], dst_v)

            def step(j, carry2):
                pltpu.sync_copy(ones_v, dacc.at[dst_v.at[j]], add=True)
                return carry2

            return lax.fori_loop(0, DCB, step, carry)

        lax.fori_loop(0, DCH, chunk, 0)
        plsc.subcore_barrier()

        @pl.when(c == 0)
        def _():
            pltpu.sync_copy(dacc.at[pl.ds(r0, NPT), :],
                            deg_o.at[0, pl.ds(r0, NPT), :])

        @pl.when(c == 1)
        def _():
            pltpu.sync_copy(dacc.at[pl.ds(r0, NPT), :],
                            deg_o.at[1, pl.ds(r0, NPT), :])

    return k(dst1, zd, ones)


def _sc_agg1(x, src1, dst1, zf):
    @functools.partial(
        pl.kernel,
        out_type=jax.ShapeDtypeStruct((NSC, NPAD, 128), jnp.float32),
        mesh=_sc_mesh(),
        scratch_types=[
            pltpu.VMEM_SHARED((NPAD, 128), jnp.float32),
            pltpu.VMEM((DCB, KB), jnp.int32),
            pltpu.VMEM((DCB, KB), jnp.int32),
            pltpu.VMEM((KB, 128), jnp.float32),
            pltpu.VMEM((KB, 128), jnp.float32),
            pltpu.VMEM((KB, 128), jnp.float32),
            pltpu.SemaphoreType.DMA,
            pltpu.SemaphoreType.DMA,
            pltpu.SemaphoreType.DMA,
        ],
    )
    def k(x_h, src_h, dst_h, zf_h, sum_o, acc, src_v, dst_v, rows0, rows1,
          rows2, sem0, sem1, sem2):
        c = lax.axis_index("c")
        s = lax.axis_index("s")
        tid = c * NT + s
        r0 = s * NPT
        pltpu.sync_copy(zf_h, acc.at[pl.ds(r0, NPT), :])
        plsc.subcore_barrier()

        rows = (rows0, rows1, rows2)
        sems = (sem0, sem1, sem2)
        nbuf = len(rows)
        depth = nbuf - 1

        def issue(j, b):
            pltpu.async_copy(x_h.at[src_v.at[j]], rows[b], sems[b])

        def drain_scatter(j, b):
            pltpu.make_async_copy(x_h.at[src_v.at[j]], rows[b],
                                  sems[b]).wait()
            pltpu.sync_copy(rows[b], acc.at[dst_v.at[j]], add=True)

        def chunk(ch, carry):
            pltpu.sync_copy(src_h.at[tid, ch], src_v)
            pltpu.sync_copy(dst_h.at[tid, ch], dst_v)
            for j in range(depth):
                issue(j, j % nbuf)
            for j in range(DCB):
                if j + depth < DCB:
                    issue(j + depth, (j + depth) % nbuf)
                drain_scatter(j, j % nbuf)
            return carry

        lax.fori_loop(0, DCH, chunk, 0)
        plsc.subcore_barrier()

        @pl.when(c == 0)
        def _():
            pltpu.sync_copy(acc.at[pl.ds(r0, NPT), :],
                            sum_o.at[0, pl.ds(r0, NPT), :])

        @pl.when(c == 1)
        def _():
            pltpu.sync_copy(acc.at[pl.ds(r0, NPT), :],
                            sum_o.at[1, pl.ds(r0, NPT), :])

    return k(x, src1, dst1, zf)


def _sc_agg23(h_parts, src2, dst2, zf):
    @functools.partial(
        pl.kernel,
        out_type=jax.ShapeDtypeStruct((NSC, NPAD, 128), jnp.float32),
        mesh=_sc_mesh(),
        scratch_types=[
            pltpu.VMEM_SHARED((NPAD, 128), jnp.float32),
            pltpu.VMEM((ACB, KB), jnp.int32),
            pltpu.VMEM((ACB, KB), jnp.int32),
            pltpu.VMEM((KB, 128), jnp.float32),
            pltpu.VMEM((KB, 128), jnp.float32),
            pltpu.VMEM((KB, 128), jnp.float32),
            pltpu.SemaphoreType.DMA,
            pltpu.SemaphoreType.DMA,
            pltpu.SemaphoreType.DMA,
        ],
    )
    def k(hp_h, src_h, dst_h, zf_h, sum_o, acc, src_v, dst_v, rows0, rows1,
          rows2, sem0, sem1, sem2):
        c = lax.axis_index("c")
        s = lax.axis_index("s")
        r0 = s * NPT
        pltpu.sync_copy(zf_h, acc.at[pl.ds(r0, NPT), :])
        plsc.subcore_barrier()

        rows = (rows0, rows1, rows2)
        sems = (sem0, sem1, sem2)
        nbuf = len(rows)
        depth = nbuf - 1

        def issue(j, b):
            @pl.when(c == 0)
            def _():
                pltpu.async_copy(hp_h.at[0].at[src_v.at[j]], rows[b],
                                 sems[b])

            @pl.when(c == 1)
            def _():
                pltpu.async_copy(hp_h.at[1].at[src_v.at[j]], rows[b],
                                 sems[b])

        def drain_scatter(j, b):
            pltpu.make_async_copy(hp_h.at[0].at[src_v.at[j]], rows[b],
                                  sems[b]).wait()
            pltpu.sync_copy(rows[b], acc.at[dst_v.at[j]], add=True)

        def chunk(ch, carry):
            pltpu.sync_copy(src_h.at[s, ch], src_v)
            pltpu.sync_copy(dst_h.at[s, ch], dst_v)
            for j in range(depth):
                issue(j, j % nbuf)
            for j in range(ACB):
                if j + depth < ACB:
                    issue(j + depth, (j + depth) % nbuf)
                drain_scatter(j, j % nbuf)
            return carry

        lax.fori_loop(0, ACH, chunk, 0)
        plsc.subcore_barrier()

        @pl.when(c == 0)
        def _():
            pltpu.sync_copy(acc.at[pl.ds(r0, NPT), :],
                            sum_o.at[0, pl.ds(r0, NPT), :])

        @pl.when(c == 1)
        def _():
            pltpu.sync_copy(acc.at[pl.ds(r0, NPT), :],
                            sum_o.at[1, pl.ds(r0, NPT), :])

    return k(h_parts, src2, dst2, zf)


def _inv_deg(dp):
    d = dp[0, :, 0:1] + dp[1, :, 0:1]
    return 1.0 / jnp.maximum(d, 1.0)


def _tc_layer1(sum_parts, deg_parts, x, wl, bl, wr):
    def body(sp, dp, xr, wl_r, bl_r, wr_r, out):
        mean = (sp[0] + sp[1]) * _inv_deg(dp)
        h = _dot_t(mean, wl_r[...]) + _dot_t(xr[...], wr_r[...]) + bl_r[...]
        h = _leaky(h)
        out[0] = h[:, :128]
        out[1] = h[:, 128:]

    return pl.pallas_call(
        body,
        grid=(NB,),
        in_specs=[
            pl.BlockSpec((NSC, RB, 128), lambda i: (0, i, 0)),
            pl.BlockSpec((NSC, RB, 128), lambda i: (0, i, 0)),
            pl.BlockSpec((RB, 128), lambda i: (i, 0)),
            pl.BlockSpec((256, 128), lambda i: (0, 0)),
            pl.BlockSpec((1, 256), lambda i: (0, 0)),
            pl.BlockSpec((256, 128), lambda i: (0, 0)),
        ],
        out_specs=pl.BlockSpec((NSC, RB, 128), lambda i: (0, i, 0)),
        out_shape=jax.ShapeDtypeStruct((NSC, N, 128), jnp.float32),
    )(sum_parts, deg_parts, x, wl, bl, wr)


def _tc_layer2(sum_parts, deg_parts, h_parts, wl, bl, wr):
    def body(sp, dp, hp, wl_r, bl_r, wr_r, out):
        mean = jnp.concatenate([sp[0], sp[1]], axis=1) * _inv_deg(dp)
        selfv = jnp.concatenate([hp[0], hp[1]], axis=1)
        h = _dot_t(mean, wl_r[...]) + _dot_t(selfv, wr_r[...]) + bl_r[...]
        h = _leaky(h)
        out[0] = h[:, :128]
        out[1] = h[:, 128:]

    return pl.pallas_call(
        body,
        grid=(NB,),
        in_specs=[
            pl.BlockSpec((NSC, RB, 128), lambda i: (0, i, 0)),
            pl.BlockSpec((NSC, RB, 128), lambda i: (0, i, 0)),
            pl.BlockSpec((NSC, RB, 128), lambda i: (0, i, 0)),
            pl.BlockSpec((256, 256), lambda i: (0, 0)),
            pl.BlockSpec((1, 256), lambda i: (0, 0)),
            pl.BlockSpec((256, 256), lambda i: (0, 0)),
        ],
        out_specs=pl.BlockSpec((NSC, RB, 128), lambda i: (0, i, 0)),
        out_shape=jax.ShapeDtypeStruct((NSC, N, 128), jnp.float32),
    )(sum_parts, deg_parts, h_parts, wl, bl, wr)


def _tc_layer3_pool_mlp(sum_parts, deg_parts, h_parts, wl, bl, wr, batch,
                        wg, bg, wf1, bf1, wf2, bf2):
    def body(sp, dp, hp, wl_r, bl_r, wr_r, batch_s, wg_r, bg_r,
             wf1_r, bf1_r, wf2_r, bf2_r, out, g_acc, h3_v):
        i = pl.program_id(0)

        @pl.when(i == 0)
        def _():
            g_acc[...] = jnp.full((G, 512), -jnp.inf, jnp.float32)

        mean = jnp.concatenate([sp[0], sp[1]], axis=1) * _inv_deg(dp)
        selfv = jnp.concatenate([hp[0], hp[1]], axis=1)
        h = _dot_t(mean, wl_r[...]) + _dot_t(selfv, wr_r[...]) + bl_r[...]
        h3_v[...] = _leaky(h)

        def row_step(r, carry):
            b_i = batch_s[i * RB + r]
            row = h3_v[pl.ds(r, 1), :]
            cur = g_acc[pl.ds(b_i, 1), :]
            g_acc[pl.ds(b_i, 1), :] = jnp.maximum(cur, row)
            return carry

        lax.fori_loop(0, RB, row_step, 0)

        @pl.when(i == NB - 1)
        def _():
            g = g_acc[...]
            t = _dot_t(g, wg_r[...]) + bg_r[...]
            t = _leaky(_dot_t(t, wf1_r[...]) + bf1_r[...])
            out[...] = _dot_t(t, wf2_r[...]) + bf2_r[...]

    return pl.pallas_call(
        body,
        grid=(NB,),
        in_specs=[
            pl.BlockSpec((NSC, RB, 128), lambda i: (0, i, 0)),
            pl.BlockSpec((NSC, RB, 128), lambda i: (0, i, 0)),
            pl.BlockSpec((NSC, RB, 128), lambda i: (0, i, 0)),
            pl.BlockSpec((512, 256), lambda i: (0, 0)),
            pl.BlockSpec((1, 512), lambda i: (0, 0)),
            pl.BlockSpec((512, 256), lambda i: (0, 0)),
            pl.BlockSpec((N,), lambda i: (0,), memory_space=pltpu.SMEM),
            pl.BlockSpec((128, 512), lambda i: (0, 0)),
            pl.BlockSpec((1, 128), lambda i: (0, 0)),
            pl.BlockSpec((128, 128), lambda i: (0, 0)),
            pl.BlockSpec((1, 128), lambda i: (0, 0)),
            pl.BlockSpec((2, 128), lambda i: (0, 0)),
            pl.BlockSpec((1, 2), lambda i: (0, 0)),
        ],
        out_specs=pl.BlockSpec((G, 2), lambda i: (0, 0)),
        out_shape=jax.ShapeDtypeStruct((G, 2), jnp.float32),
        scratch_shapes=[
            pltpu.VMEM((G, 512), jnp.float32),
            pltpu.VMEM((RB, 512), jnp.float32),
        ],
    )(sum_parts, deg_parts, h_parts, wl, bl, wr, batch,
      wg, bg, wf1, bf1, wf2, bf2)


def kernel(x, edge_index, batch, W_l1, b_l1, W_r1, W_l2, b_l2, W_r2,
           W_l3, b_l3, W_r3, W_g1, b_g1, W_f1, b_f1, W_f2, b_f2):
    src1 = edge_index[0].reshape(NSC * NT, DCH, DCB, KB)
    dst1 = edge_index[1].reshape(NSC * NT, DCH, DCB, KB)
    srcf = edge_index[0].reshape(NT, ACH, ACB, KB)
    dstf = edge_index[1].reshape(NT, ACH, ACB, KB)
    zf = jnp.zeros((NPT, 128), jnp.float32)
    ones = jnp.ones((KB, 128), jnp.float32)

    deg = _sc_deg(dst1, zf, ones)
    sum1 = _sc_agg1(x, src1, dst1, zf)
    h1 = _tc_layer1(sum1, deg, x, W_l1, b_l1.reshape(1, -1), W_r1)
    sum2 = _sc_agg23(h1, srcf, dstf, zf)
    h2 = _tc_layer2(sum2, deg, h1, W_l2, b_l2.reshape(1, -1), W_r2)
    sum3 = _sc_agg23(h2, srcf, dstf, zf)
    out = _tc_layer3_pool_mlp(
        sum3, deg, h2, W_l3, b_l3.reshape(1, -1), W_r3, batch,
        W_g1, b_g1.reshape(1, -1), W_f1, b_f1.reshape(1, -1),
        W_f2, b_f2.reshape(1, -1))
    return out


# speedup vs baseline: 7.9570x; 1.1874x over previous
import functools

import jax
import jax.numpy as jnp
from jax import lax
from jax.experimental import pallas as pl
from jax.experimental.pallas import tpu as pltpu
from jax.experimental.pallas import tpu_sc as plsc

N = 10000
E = 320000
G = 256
NEG = 0.01

KB = 80
ER = E // KB
NSC = 2
NT = 16
NPAD = 10240
NPT = NPAD // NT
RPT1 = ER // NSC // NT
DCH = 25
DCB = RPT1 // DCH
RPT23 = ER // NT
ACH = 10
ACB = RPT23 // ACH
A1CH = 5
A1CB = RPT1 // A1CH

RB = 2000
NB = N // RB


def _leaky(v):
    return jnp.where(v > 0, v, NEG * v)


def _dot_t(a, w):
    return lax.dot_general(a, w, (((1,), (1,)), ((), ())),
                           preferred_element_type=jnp.float32)


def _sc_mesh():
    return plsc.VectorSubcoreMesh(core_axis_name="c", subcore_axis_name="s")


def _sc_deg(dst1, zd, ones):
    @functools.partial(
        pl.kernel,
        out_type=jax.ShapeDtypeStruct((NSC, NPAD, 128), jnp.float32),
        mesh=_sc_mesh(),
        scratch_types=[
            pltpu.VMEM_SHARED((NPAD, 128), jnp.float32),
            pltpu.VMEM((DCB, KB), jnp.int32),
            pltpu.VMEM((KB, 128), jnp.float32),
        ],
    )
    def k(dst_h, zd_h, on_h, deg_o, dacc, dst_v, ones_v):
        c = lax.axis_index("c")
        s = lax.axis_index("s")
        r0 = s * NPT
        pltpu.sync_copy(zd_h, dacc.at[pl.ds(r0, NPT), :])
        pltpu.sync_copy(on_h, ones_v)
        tid = c * NT + s
        plsc.subcore_barrier()

        def chunk(ch, carry):
            pltpu.sync_copy(dst_h.at[tid, ch], dst_v)

            def step(j, carry2):
                pltpu.sync_copy(ones_v, dacc.at[dst_v.at[j]], add=True)
                return carry2

            return lax.fori_loop(0, DCB, step, carry)

        lax.fori_loop(0, DCH, chunk, 0)
        plsc.subcore_barrier()

        @pl.when(c == 0)
        def _():
            pltpu.sync_copy(dacc.at[pl.ds(r0, NPT), :],
                            deg_o.at[0, pl.ds(r0, NPT), :])

        @pl.when(c == 1)
        def _():
            pltpu.sync_copy(dacc.at[pl.ds(r0, NPT), :],
                            deg_o.at[1, pl.ds(r0, NPT), :])

    return k(dst1, zd, ones)


def _sc_agg1(x, idx1, zf):
    @functools.partial(
        pl.kernel,
        out_type=jax.ShapeDtypeStruct((NSC, NPAD, 128), jnp.float32),
        mesh=_sc_mesh(),
        scratch_types=[
            pltpu.VMEM_SHARED((NPAD, 128), jnp.float32),
            pltpu.VMEM((2, A1CB, KB), jnp.int32),
            pltpu.VMEM((KB, 128), jnp.float32),
            pltpu.VMEM((KB, 128), jnp.float32),
            pltpu.VMEM((KB, 128), jnp.float32),
            pltpu.SemaphoreType.DMA,
            pltpu.SemaphoreType.DMA,
            pltpu.SemaphoreType.DMA,
        ],
    )
    def k(x_h, idx_h, zf_h, sum_o, acc, idx_v, rows0, rows1,
          rows2, sem0, sem1, sem2):
        c = lax.axis_index("c")
        s = lax.axis_index("s")
        tid = c * NT + s
        r0 = s * NPT
        pltpu.sync_copy(zf_h, acc.at[pl.ds(r0, NPT), :])
        plsc.subcore_barrier()

        rows = (rows0, rows1, rows2)
        sems = (sem0, sem1, sem2)
        nbuf = len(rows)
        depth = nbuf - 1

        def issue(j, b):
            pltpu.async_copy(x_h.at[idx_v.at[0, j]], rows[b], sems[b])

        def drain_scatter(j, b):
            pltpu.make_async_copy(x_h.at[idx_v.at[0, j]], rows[b],
                                  sems[b]).wait()
            pltpu.sync_copy(rows[b], acc.at[idx_v.at[1, j]], add=True)

        def chunk(ch, carry):
            pltpu.sync_copy(idx_h.at[tid, ch], idx_v)
            for j in range(depth):
                issue(j, j % nbuf)
            for j in range(A1CB):
                if j + depth < A1CB:
                    issue(j + depth, (j + depth) % nbuf)
                drain_scatter(j, j % nbuf)
            return carry

        lax.fori_loop(0, A1CH, chunk, 0)
        plsc.subcore_barrier()

        @pl.when(c == 0)
        def _():
            pltpu.sync_copy(acc.at[pl.ds(r0, NPT), :],
                            sum_o.at[0, pl.ds(r0, NPT), :])

        @pl.when(c == 1)
        def _():
            pltpu.sync_copy(acc.at[pl.ds(r0, NPT), :],
                            sum_o.at[1, pl.ds(r0, NPT), :])

    return k(x, idx1, zf)


def _sc_agg23(h_parts, idxf, zf):
    @functools.partial(
        pl.kernel,
        out_type=jax.ShapeDtypeStruct((NSC, NPAD, 128), jnp.float32),
        mesh=_sc_mesh(),
        scratch_types=[
            pltpu.VMEM_SHARED((NPAD, 128), jnp.float32),
            pltpu.VMEM((2, ACB, KB), jnp.int32),
            pltpu.VMEM((KB, 128), jnp.float32),
            pltpu.VMEM((KB, 128), jnp.float32),
            pltpu.VMEM((KB, 128), jnp.float32),
            pltpu.SemaphoreType.DMA,
            pltpu.SemaphoreType.DMA,
            pltpu.SemaphoreType.DMA,
        ],
    )
    def k(hp_h, idx_h, zf_h, sum_o, acc, idx_v, rows0, rows1,
          rows2, sem0, sem1, sem2):
        c = lax.axis_index("c")
        s = lax.axis_index("s")
        r0 = s * NPT
        pltpu.sync_copy(zf_h, acc.at[pl.ds(r0, NPT), :])
        plsc.subcore_barrier()

        rows = (rows0, rows1, rows2)
        sems = (sem0, sem1, sem2)
        nbuf = len(rows)
        depth = nbuf - 1

        def issue(j, b):
            @pl.when(c == 0)
            def _():
                pltpu.async_copy(hp_h.at[0].at[idx_v.at[0, j]], rows[b],
                                 sems[b])

            @pl.when(c == 1)
            def _():
                pltpu.async_copy(hp_h.at[1].at[idx_v.at[0, j]], rows[b],
                                 sems[b])

        def drain_scatter(j, b):
            pltpu.make_async_copy(hp_h.at[0].at[idx_v.at[0, j]], rows[b],
                                  sems[b]).wait()
            pltpu.sync_copy(rows[b], acc.at[idx_v.at[1, j]], add=True)

        def chunk(ch, carry):
            pltpu.sync_copy(idx_h.at[s, ch], idx_v)
            for j in range(depth):
                issue(j, j % nbuf)
            for j in range(ACB):
                if j + depth < ACB:
                    issue(j + depth, (j + depth) % nbuf)
                drain_scatter(j, j % nbuf)
            return carry

        lax.fori_loop(0, ACH, chunk, 0)
        plsc.subcore_barrier()

        @pl.when(c == 0)
        def _():
            pltpu.sync_copy(acc.at[pl.ds(r0, NPT), :],
                            sum_o.at[0, pl.ds(r0, NPT), :])

        @pl.when(c == 1)
        def _():
            pltpu.sync_copy(acc.at[pl.ds(r0, NPT), :],
                            sum_o.at[1, pl.ds(r0, NPT), :])

    return k(h_parts, idxf, zf)


def _inv_deg(dp):
    d = dp[0, :, 0:1] + dp[1, :, 0:1]
    return 1.0 / jnp.maximum(d, 1.0)


def _tc_layer1(sum_parts, deg_parts, x, wl, bl, wr):
    def body(sp, dp, xr, wl_r, bl_r, wr_r, out):
        mean = (sp[0] + sp[1]) * _inv_deg(dp)
        h = _dot_t(mean, wl_r[...]) + _dot_t(xr[...], wr_r[...]) + bl_r[...]
        h = _leaky(h)
        out[0] = h[:, :128]
        out[1] = h[:, 128:]

    return pl.pallas_call(
        body,
        grid=(NB,),
        in_specs=[
            pl.BlockSpec((NSC, RB, 128), lambda i: (0, i, 0)),
            pl.BlockSpec((NSC, RB, 128), lambda i: (0, i, 0)),
            pl.BlockSpec((RB, 128), lambda i: (i, 0)),
            pl.BlockSpec((256, 128), lambda i: (0, 0)),
            pl.BlockSpec((1, 256), lambda i: (0, 0)),
            pl.BlockSpec((256, 128), lambda i: (0, 0)),
        ],
        out_specs=pl.BlockSpec((NSC, RB, 128), lambda i: (0, i, 0)),
        out_shape=jax.ShapeDtypeStruct((NSC, N, 128), jnp.float32),
    )(sum_parts, deg_parts, x, wl, bl, wr)


def _tc_layer2(sum_parts, deg_parts, h_parts, wl, bl, wr):
    def body(sp, dp, hp, wl_r, bl_r, wr_r, out):
        mean = jnp.concatenate([sp[0], sp[1]], axis=1) * _inv_deg(dp)
        selfv = jnp.concatenate([hp[0], hp[1]], axis=1)
        h = _dot_t(mean, wl_r[...]) + _dot_t(selfv, wr_r[...]) + bl_r[...]
        h = _leaky(h)
        out[0] = h[:, :128]
        out[1] = h[:, 128:]

    return pl.pallas_call(
        body,
        grid=(NB,),
        in_specs=[
            pl.BlockSpec((NSC, RB, 128), lambda i: (0, i, 0)),
            pl.BlockSpec((NSC, RB, 128), lambda i: (0, i, 0)),
            pl.BlockSpec((NSC, RB, 128), lambda i: (0, i, 0)),
            pl.BlockSpec((256, 256), lambda i: (0, 0)),
            pl.BlockSpec((1, 256), lambda i: (0, 0)),
            pl.BlockSpec((256, 256), lambda i: (0, 0)),
        ],
        out_specs=pl.BlockSpec((NSC, RB, 128), lambda i: (0, i, 0)),
        out_shape=jax.ShapeDtypeStruct((NSC, N, 128), jnp.float32),
    )(sum_parts, deg_parts, h_parts, wl, bl, wr)


def _tc_layer3_pool_mlp(sum_parts, deg_parts, h_parts, wl, bl, wr, batch,
                        wg, bg, wf1, bf1, wf2, bf2):
    def body(sp, dp, hp, wl_r, bl_r, wr_r, batch_s, wg_r, bg_r,
             wf1_r, bf1_r, wf2_r, bf2_r, out, g_acc, h3_v):
        i = pl.program_id(0)

        @pl.when(i == 0)
        def _():
            g_acc[...] = jnp.full((G, 512), -jnp.inf, jnp.float32)

        mean = jnp.concatenate([sp[0], sp[1]], axis=1) * _inv_deg(dp)
        selfv = jnp.concatenate([hp[0], hp[1]], axis=1)
        h = _dot_t(mean, wl_r[...]) + _dot_t(selfv, wr_r[...]) + bl_r[...]
        h3_v[...] = _leaky(h)

        def row_step(r, carry):
            b_i = batch_s[i * RB + r]
            row = h3_v[pl.ds(r, 1), :]
            cur = g_acc[pl.ds(b_i, 1), :]
            g_acc[pl.ds(b_i, 1), :] = jnp.maximum(cur, row)
            return carry

        lax.fori_loop(0, RB, row_step, 0)

        @pl.when(i == NB - 1)
        def _():
            g = g_acc[...]
            t = _dot_t(g, wg_r[...]) + bg_r[...]
            t = _leaky(_dot_t(t, wf1_r[...]) + bf1_r[...])
            out[...] = _dot_t(t, wf2_r[...]) + bf2_r[...]

    return pl.pallas_call(
        body,
        grid=(NB,),
        in_specs=[
            pl.BlockSpec((NSC, RB, 128), lambda i: (0, i, 0)),
            pl.BlockSpec((NSC, RB, 128), lambda i: (0, i, 0)),
            pl.BlockSpec((NSC, RB, 128), lambda i: (0, i, 0)),
            pl.BlockSpec((512, 256), lambda i: (0, 0)),
            pl.BlockSpec((1, 512), lambda i: (0, 0)),
            pl.BlockSpec((512, 256), lambda i: (0, 0)),
            pl.BlockSpec((N,), lambda i: (0,), memory_space=pltpu.SMEM),
            pl.BlockSpec((128, 512), lambda i: (0, 0)),
            pl.BlockSpec((1, 128), lambda i: (0, 0)),
            pl.BlockSpec((128, 128), lambda i: (0, 0)),
            pl.BlockSpec((1, 128), lambda i: (0, 0)),
            pl.BlockSpec((2, 128), lambda i: (0, 0)),
            pl.BlockSpec((1, 2), lambda i: (0, 0)),
        ],
        out_specs=pl.BlockSpec((G, 2), lambda i: (0, 0)),
        out_shape=jax.ShapeDtypeStruct((G, 2), jnp.float32),
        scratch_shapes=[
            pltpu.VMEM((G, 512), jnp.float32),
            pltpu.VMEM((RB, 512), jnp.float32),
        ],
    )(sum_parts, deg_parts, h_parts, wl, bl, wr, batch,
      wg, bg, wf1, bf1, wf2, bf2)


def kernel(x, edge_index, batch, W_l1, b_l1, W_r1, W_l2, b_l2, W_r2,
           W_l3, b_l3, W_r3, W_g1, b_g1, W_f1, b_f1, W_f2, b_f2):
    dst1 = edge_index[1].reshape(NSC * NT, DCH, DCB, KB)
    idx1 = jnp.stack([edge_index[0].reshape(NSC * NT, A1CH, A1CB, KB),
                      edge_index[1].reshape(NSC * NT, A1CH, A1CB, KB)],
                     axis=2)
    idxf = jnp.stack([edge_index[0].reshape(NT, ACH, ACB, KB),
                      edge_index[1].reshape(NT, ACH, ACB, KB)], axis=2)
    zf = jnp.zeros((NPT, 128), jnp.float32)
    ones = jnp.ones((KB, 128), jnp.float32)

    deg = _sc_deg(dst1, zf, ones)
    sum1 = _sc_agg1(x, idx1, zf)
    h1 = _tc_layer1(sum1, deg, x, W_l1, b_l1.reshape(1, -1), W_r1)
    sum2 = _sc_agg23(h1, idxf, zf)
    h2 = _tc_layer2(sum2, deg, h1, W_l2, b_l2.reshape(1, -1), W_r2)
    sum3 = _sc_agg23(h2, idxf, zf)
    out = _tc_layer3_pool_mlp(
        sum3, deg, h2, W_l3, b_l3.reshape(1, -1), W_r3, batch,
        W_g1, b_g1.reshape(1, -1), W_f1, b_f1.reshape(1, -1),
        W_f2, b_f2.reshape(1, -1))
    return out
